# Initial kernel scaffold; baseline (speedup 1.0000x reference)
import jax
import jax.numpy as jnp
from jax.experimental import pallas as pl


def kernel(x):
    raise NotImplementedError("write your pallas kernel here")


# speedup vs baseline: 28.8724x; 28.8724x over previous
import functools

import jax
import jax.numpy as jnp
from jax import lax
from jax.experimental import pallas as pl
from jax.experimental.pallas import tpu as pltpu
from jax.experimental.pallas import tpu_sc as plsc

K = 8
B, S, C = 4, 8192, 1024
L = 16
NC, NS = 2, 16
NW = NC * NS
CPW = (B * C) // NW
NCHUNK = CPW // L
WIN = 256
NWIN = S // WIN

NEG_INF = float("-inf")


def _insert_rows(buf, j, r):
    def body(s, carry):
        v = buf[s, pl.ds(j * L, L)]
        out = []
        for k in range(K):
            rk = carry[k]
            out.append(jnp.maximum(rk, v))
            v = jnp.minimum(rk, v)
        return tuple(out)

    return lax.fori_loop(0, WIN, body, r, unroll=False)


def kernel(x):
    mesh = plsc.VectorSubcoreMesh(core_axis_name="c", subcore_axis_name="s")

    @functools.partial(
        pl.kernel,
        out_type=jax.ShapeDtypeStruct((B, C * K), jnp.float32),
        mesh=mesh,
        scratch_types=[
            pltpu.VMEM((WIN, CPW), jnp.float32),
            pltpu.VMEM((K, CPW), jnp.float32),
            pltpu.VMEM((K * CPW,), jnp.float32),
        ],
    )
    def run(x_hbm, out_hbm, buf, rbuf, obuf):
        wid = lax.axis_index("s") * NC + lax.axis_index("c")
        b = wid // (C // CPW)
        c0 = (wid % (C // CPW)) * CPW

        for j in range(NCHUNK):
            for k in range(K):
                rbuf[k, pl.ds(j * L, L)] = jnp.full((L,), NEG_INF)

        @pl.loop(0, NWIN)
        def _window(w):
            pltpu.sync_copy(
                x_hbm.at[b, pl.ds(w * WIN, WIN), pl.ds(c0, CPW)], buf
            )
            for j in range(NCHUNK):
                r = tuple(rbuf[k, pl.ds(j * L, L)] for k in range(K))
                r = _insert_rows(buf, j, r)
                for k in range(K):
                    rbuf[k, pl.ds(j * L, L)] = r[k]

        lane = lax.iota(jnp.int32, L)
        kmask = [(lane & (K - 1)) == k for k in range(K)]
        for t in range(CPW * K // L):
            ch0 = 2 * t
            j = ch0 // L
            m = ch0 % L
            idx = jnp.where(lane < K, m, m + 1)
            out = jnp.full((L,), NEG_INF)
            for k in range(K):
                g = jnp.take(rbuf[k, pl.ds(j * L, L)], idx)
                out = jnp.where(kmask[k], g, out)
            obuf[pl.ds(t * L, L)] = out
        pltpu.sync_copy(obuf, out_hbm.at[b, pl.ds(c0 * K, CPW * K)])

    return run(x)


# speedup vs baseline: 61.6087x; 2.1338x over previous
import functools

import jax
import jax.numpy as jnp
from jax import lax
from jax.experimental import pallas as pl
from jax.experimental.pallas import tpu as pltpu
from jax.experimental.pallas import tpu_sc as plsc

K = 8
B, S, C = 4, 8192, 1024
L = 16
NC, NS = 2, 16
NW = NC * NS
CPW = (B * C) // NW
NCHUNK = CPW // L
WIN = 256
NWIN = S // WIN

NEG_INF = float("-inf")

SORT8 = [
    (0, 1), (2, 3), (4, 5), (6, 7),
    (0, 2), (1, 3), (4, 6), (5, 7),
    (1, 2), (5, 6),
    (0, 4), (1, 5), (2, 6), (3, 7),
    (2, 4), (3, 5),
    (1, 2), (3, 4), (5, 6),
]
BITONIC8 = [
    (0, 4), (1, 5), (2, 6), (3, 7),
    (0, 2), (1, 3), (4, 6), (5, 7),
    (0, 1), (2, 3), (4, 5), (6, 7),
]


def _apply_net(v, net):
    v = list(v)
    for a, b in net:
        hi = jnp.maximum(v[a], v[b])
        lo = jnp.minimum(v[a], v[b])
        v[a], v[b] = hi, lo
    return v


def _merge_top8(r, c):
    z = [jnp.maximum(r[i], c[K - 1 - i]) for i in range(K)]
    return _apply_net(z, BITONIC8)


def _process_window(buf, rbuf):
    for j in range(NCHUNK):
        r = tuple(rbuf[k, pl.ds(j * L, L)] for k in range(K))

        def body(i, r, j=j):
            c = [buf[i * K + t, pl.ds(j * L, L)] for t in range(K)]
            c = _apply_net(c, SORT8)
            return tuple(_merge_top8(list(r), c))

        r = lax.fori_loop(0, WIN // K, body, r, unroll=2)
        for k in range(K):
            rbuf[k, pl.ds(j * L, L)] = r[k]


def kernel(x):
    mesh = plsc.VectorSubcoreMesh(core_axis_name="c", subcore_axis_name="s")

    @functools.partial(
        pl.kernel,
        out_type=jax.ShapeDtypeStruct((B, C * K), jnp.float32),
        mesh=mesh,
        scratch_types=[
            pltpu.VMEM((WIN, CPW), jnp.float32),
            pltpu.VMEM((WIN, CPW), jnp.float32),
            pltpu.VMEM((K, CPW), jnp.float32),
            pltpu.VMEM((K * CPW,), jnp.float32),
            pltpu.SemaphoreType.DMA,
            pltpu.SemaphoreType.DMA,
        ],
    )
    def run(x_hbm, out_hbm, buf0, buf1, rbuf, obuf, sem0, sem1):
        wid = lax.axis_index("s") * NC + lax.axis_index("c")
        b = wid // (C // CPW)
        c0 = (wid % (C // CPW)) * CPW

        def src(w):
            return x_hbm.at[b, pl.ds(w * WIN, WIN), pl.ds(c0, CPW)]

        for j in range(NCHUNK):
            for k in range(K):
                rbuf[k, pl.ds(j * L, L)] = jnp.full((L,), NEG_INF)

        pltpu.async_copy(src(0), buf0, sem0)

        @pl.loop(0, NWIN // 2)
        def _pair(p):
            w0 = 2 * p
            pltpu.async_copy(src(w0 + 1), buf1, sem1)
            pltpu.make_async_copy(src(0), buf0, sem0).wait()
            _process_window(buf0, rbuf)

            @pl.when(w0 + 2 < NWIN)
            def _():
                pltpu.async_copy(src(w0 + 2), buf0, sem0)

            pltpu.make_async_copy(src(0), buf1, sem1).wait()
            _process_window(buf1, rbuf)

        lane = lax.iota(jnp.int32, L)
        kmask = [(lane & (K - 1)) == k for k in range(K)]
        for t in range(CPW * K // L):
            ch0 = 2 * t
            j = ch0 // L
            m = ch0 % L
            idx = jnp.where(lane < K, m, m + 1)
            out = jnp.full((L,), NEG_INF)
            for k in range(K):
                g = jnp.take(rbuf[k, pl.ds(j * L, L)], idx)
                out = jnp.where(kmask[k], g, out)
            obuf[pl.ds(t * L, L)] = out
        pltpu.sync_copy(obuf, out_hbm.at[b, pl.ds(c0 * K, CPW * K)])

    return run(x)
